# baseline (speedup 1.0000x reference)
import functools

import jax
import jax.numpy as jnp
from jax import lax
from jax.experimental import pallas as pl
from jax.experimental.pallas import tpu as pltpu
from jax.experimental.pallas import tpu_sc as plsc

B, L, D = 1024, 200, 128
BB = 128

_info = plsc.get_sparse_core_info()
_NC, _NS = _info.num_cores, _info.num_subcores
_NW = _NC * _NS
_BPW = B // _NW


def _sc_gather(table_hbm, idx_hbm, out_hbm, idx_v, rows_v, sem):
    wid = lax.axis_index("s") * _NC + lax.axis_index("c")
    base = wid * _BPW
    pltpu.sync_copy(idx_hbm.at[pl.ds(base, _BPW)], idx_v)
    pltpu.async_copy(table_hbm.at[idx_v], rows_v, sem).wait()
    pltpu.sync_copy(rows_v, out_hbm.at[pl.ds(base, _BPW)])


def _copy_x_body(x_ref, out_ref):
    out_ref[...] = x_ref[...]


BBK2 = 32


def _emb_body(_, rows_ref, out_ref):
    out_ref[...] = jnp.broadcast_to(rows_ref[...][:, None, :], (BBK2, L, D))


def kernel(x, labels_pointer, emb_table):
    gather = functools.partial(
        pl.kernel,
        mesh=plsc.VectorSubcoreMesh(core_axis_name="c", subcore_axis_name="s"),
        out_type=jax.ShapeDtypeStruct((B, D), jnp.float32),
        scratch_types=[
            pltpu.VMEM((_BPW,), jnp.int32),
            pltpu.VMEM((_BPW, D), jnp.float32),
            pltpu.SemaphoreType.DMA,
        ],
    )(_sc_gather)
    rows = gather(emb_table, labels_pointer)

    out1 = pl.pallas_call(
        _copy_x_body,
        grid=(B // BB,),
        in_specs=[pl.BlockSpec((BB, L, D), lambda i: (i, 0, 0))],
        out_specs=pl.BlockSpec((BB, L, D), lambda i: (i, 0, 0)),
        out_shape=jax.ShapeDtypeStruct((B, L, 2 * D), x.dtype),
        compiler_params=pltpu.CompilerParams(
            dimension_semantics=("parallel",),
        ),
    )(x)

    return pl.pallas_call(
        _emb_body,
        grid=(B // BBK2,),
        in_specs=[
            pl.BlockSpec(memory_space=pl.ANY),
            pl.BlockSpec((BBK2, D), lambda i: (i, 0)),
        ],
        out_specs=pl.BlockSpec((BBK2, L, D), lambda i: (i, 0, 1)),
        out_shape=jax.ShapeDtypeStruct((B, L, 2 * D), x.dtype),
        input_output_aliases={0: 0},
        compiler_params=pltpu.CompilerParams(
            dimension_semantics=("parallel",),
        ),
    )(out1, rows)


# speedup vs baseline: 1.0231x; 1.0231x over previous
import functools

import jax
import jax.numpy as jnp
from jax import lax
from jax.experimental import pallas as pl
from jax.experimental.pallas import tpu as pltpu
from jax.experimental.pallas import tpu_sc as plsc

B, L, D = 1024, 200, 128
BB = 64

_info = plsc.get_sparse_core_info()
_NC, _NS = _info.num_cores, _info.num_subcores
_NW = _NC * _NS
_BPW = B // _NW


def _sc_gather(table_hbm, idx_hbm, out_hbm, idx_v, rows_v, sem):
    wid = lax.axis_index("s") * _NC + lax.axis_index("c")
    base = wid * _BPW
    pltpu.sync_copy(idx_hbm.at[pl.ds(base, _BPW)], idx_v)
    pltpu.async_copy(table_hbm.at[idx_v], rows_v, sem).wait()
    pltpu.sync_copy(rows_v, out_hbm.at[pl.ds(base, _BPW)])


def _copy_x_body(x_ref, out_ref):
    out_ref[...] = x_ref[...]


BBK2 = 64


def _emb_body(_, rows_ref, out_ref):
    out_ref[...] = jnp.broadcast_to(rows_ref[...][:, None, :], (BBK2, L, D))


def kernel(x, labels_pointer, emb_table):
    gather = functools.partial(
        pl.kernel,
        mesh=plsc.VectorSubcoreMesh(core_axis_name="c", subcore_axis_name="s"),
        out_type=jax.ShapeDtypeStruct((B, D), jnp.float32),
        scratch_types=[
            pltpu.VMEM((_BPW,), jnp.int32),
            pltpu.VMEM((_BPW, D), jnp.float32),
            pltpu.SemaphoreType.DMA,
        ],
    )(_sc_gather)
    rows = gather(emb_table, labels_pointer)

    out1 = pl.pallas_call(
        _copy_x_body,
        grid=(B // BB,),
        in_specs=[pl.BlockSpec((BB, L, D), lambda i: (i, 0, 0))],
        out_specs=pl.BlockSpec((BB, L, D), lambda i: (i, 0, 0)),
        out_shape=jax.ShapeDtypeStruct((B, L, 2 * D), x.dtype),
        compiler_params=pltpu.CompilerParams(
            dimension_semantics=("parallel",),
        ),
    )(x)

    return pl.pallas_call(
        _emb_body,
        grid=(B // BBK2,),
        in_specs=[
            pl.BlockSpec(memory_space=pl.ANY),
            pl.BlockSpec((BBK2, D), lambda i: (i, 0)),
        ],
        out_specs=pl.BlockSpec((BBK2, L, D), lambda i: (i, 0, 1)),
        out_shape=jax.ShapeDtypeStruct((B, L, 2 * D), x.dtype),
        input_output_aliases={0: 0},
        compiler_params=pltpu.CompilerParams(
            dimension_semantics=("parallel",),
        ),
    )(out1, rows)


# speedup vs baseline: 1.0294x; 1.0061x over previous
import functools

import jax
import jax.numpy as jnp
from jax import lax
from jax.experimental import pallas as pl
from jax.experimental.pallas import tpu as pltpu
from jax.experimental.pallas import tpu_sc as plsc

B, L, D = 1024, 200, 128
BB = 128

_info = plsc.get_sparse_core_info()
_NC, _NS = _info.num_cores, _info.num_subcores
_NW = _NC * _NS
_BPW = B // _NW


def _sc_gather(table_hbm, idx_hbm, out_hbm, idx_v, rows_v, sem):
    wid = lax.axis_index("s") * _NC + lax.axis_index("c")
    base = wid * _BPW
    pltpu.sync_copy(idx_hbm.at[pl.ds(base, _BPW)], idx_v)
    pltpu.async_copy(table_hbm.at[idx_v], rows_v, sem).wait()
    pltpu.sync_copy(rows_v, out_hbm.at[pl.ds(base, _BPW)])


def _copy_x_body(x_ref, out_ref):
    out_ref[...] = x_ref[...]


BBK2 = 64


def _emb_body(_, rows_ref, out_ref):
    out_ref[...] = jnp.broadcast_to(rows_ref[...][:, None, :], (BBK2, L, D))


def kernel(x, labels_pointer, emb_table):
    gather = functools.partial(
        pl.kernel,
        mesh=plsc.VectorSubcoreMesh(core_axis_name="c", subcore_axis_name="s"),
        out_type=jax.ShapeDtypeStruct((B, D), jnp.float32),
        scratch_types=[
            pltpu.VMEM((_BPW,), jnp.int32),
            pltpu.VMEM((_BPW, D), jnp.float32),
            pltpu.SemaphoreType.DMA,
        ],
    )(_sc_gather)
    rows = gather(emb_table, labels_pointer)

    out1 = pl.pallas_call(
        _copy_x_body,
        grid=(B // BB,),
        in_specs=[pl.BlockSpec((BB, L, D), lambda i: (i, 0, 0))],
        out_specs=pl.BlockSpec((BB, L, D), lambda i: (i, 0, 0)),
        out_shape=jax.ShapeDtypeStruct((B, L, 2 * D), x.dtype),
        compiler_params=pltpu.CompilerParams(
            dimension_semantics=("parallel",),
        ),
    )(x)

    return pl.pallas_call(
        _emb_body,
        grid=(B // BBK2,),
        in_specs=[
            pl.BlockSpec(memory_space=pl.ANY),
            pl.BlockSpec((BBK2, D), lambda i: (i, 0)),
        ],
        out_specs=pl.BlockSpec((BBK2, L, D), lambda i: (i, 0, 1)),
        out_shape=jax.ShapeDtypeStruct((B, L, 2 * D), x.dtype),
        input_output_aliases={0: 0},
        compiler_params=pltpu.CompilerParams(
            dimension_semantics=("parallel",),
        ),
    )(out1, rows)


# speedup vs baseline: 1.0328x; 1.0032x over previous
import functools

import jax
import jax.numpy as jnp
from jax import lax
from jax.experimental import pallas as pl
from jax.experimental.pallas import tpu as pltpu
from jax.experimental.pallas import tpu_sc as plsc

B, L, D = 1024, 200, 128
BB = 128

_info = plsc.get_sparse_core_info()
_NC, _NS = _info.num_cores, _info.num_subcores
_NW = _NC * _NS
_BPW = B // _NW


def _sc_gather(table_hbm, idx_hbm, out_hbm, idx_v, rows_v, sem):
    wid = lax.axis_index("s") * _NC + lax.axis_index("c")
    base = wid * _BPW
    pltpu.sync_copy(idx_hbm.at[pl.ds(base, _BPW)], idx_v)
    pltpu.async_copy(table_hbm.at[idx_v], rows_v, sem).wait()
    pltpu.sync_copy(rows_v, out_hbm.at[pl.ds(base, _BPW)])


def _copy_x_body(x_ref, out_ref):
    out_ref[...] = x_ref[...]


BBK2 = 64


def _emb_body(_, rows_ref, out_ref):
    out_ref[...] = jnp.broadcast_to(rows_ref[...][:, None, :], (BBK2, L, D))


def kernel(x, labels_pointer, emb_table):
    gather = functools.partial(
        pl.kernel,
        mesh=plsc.VectorSubcoreMesh(core_axis_name="c", subcore_axis_name="s"),
        out_type=jax.ShapeDtypeStruct((B, D), jnp.float32),
        scratch_types=[
            pltpu.VMEM((_BPW,), jnp.int32),
            pltpu.VMEM((_BPW, D), jnp.float32),
            pltpu.SemaphoreType.DMA,
        ],
    )(_sc_gather)
    rows = gather(emb_table, labels_pointer)

    out1 = pl.pallas_call(
        _copy_x_body,
        grid=(B // BB,),
        in_specs=[pl.BlockSpec((BB, L, D), lambda i: (i, 0, 0))],
        out_specs=pl.BlockSpec((BB, L, D), lambda i: (i, 0, 0)),
        out_shape=jax.ShapeDtypeStruct((B, L, 2 * D), x.dtype),
    )(x)

    return pl.pallas_call(
        _emb_body,
        grid=(B // BBK2,),
        in_specs=[
            pl.BlockSpec(memory_space=pl.ANY),
            pl.BlockSpec((BBK2, D), lambda i: (i, 0)),
        ],
        out_specs=pl.BlockSpec((BBK2, L, D), lambda i: (i, 0, 1)),
        out_shape=jax.ShapeDtypeStruct((B, L, 2 * D), x.dtype),
        input_output_aliases={0: 0},
    )(out1, rows)
